# baseline (speedup 1.0000x reference)
import functools

import jax
import jax.numpy as jnp
from jax import lax
from jax.experimental import pallas as pl
from jax.experimental.pallas import tpu as pltpu
from jax.experimental.pallas import tpu_sc as plsc

N = 10000
E = 320000
H = 128
XW = 16

_NC = 2
_NS = 16
_NW = _NC * _NS
_EPW = E // _NW
_NPS = N // _NS

_f32 = jnp.float32


def _silu(v):
    return v * jax.nn.sigmoid(v)


def _sc_gather_sum(A, B, row, col):
    mesh = plsc.VectorSubcoreMesh(core_axis_name="c", subcore_axis_name="s")
    CH = 400
    n_chunks = _EPW // CH

    @functools.partial(
        pl.kernel, mesh=mesh,
        out_type=jax.ShapeDtypeStruct((E, H), _f32),
        scratch_types=[
            pltpu.VMEM((CH,), jnp.int32),
            pltpu.VMEM((CH,), jnp.int32),
            pltpu.VMEM((CH, H), _f32),
            pltpu.VMEM((CH, H), _f32),
            pltpu.SemaphoreType.DMA,
            pltpu.SemaphoreType.DMA,
        ],
    )
    def k(A_hbm, B_hbm, row_hbm, col_hbm, pre_hbm,
          idxr, idxc, bufA, bufB, semA, semB):
        wid = lax.axis_index("s") * _NC + lax.axis_index("c")
        base0 = wid * _EPW

        def chunk(i, carry):
            base = base0 + i * CH
            pltpu.sync_copy(row_hbm.at[pl.ds(base, CH)], idxr)
            pltpu.sync_copy(col_hbm.at[pl.ds(base, CH)], idxc)
            cpA = pltpu.async_copy(A_hbm.at[idxr], bufA, semA)
            cpB = pltpu.async_copy(B_hbm.at[idxc], bufB, semB)
            cpA.wait()
            cpB.wait()

            def add_row(r, c2):
                for kk in range(H // 16):
                    sl = pl.ds(kk * 16, 16)
                    bufA[r, sl] = bufA[r, sl] + bufB[r, sl]
                return c2

            lax.fori_loop(0, CH, add_row, 0)
            pltpu.sync_copy(bufA, pre_hbm.at[pl.ds(base, CH)])
            return carry

        lax.fori_loop(0, n_chunks, chunk, 0)

    return k(A, B, row, col)


def _sc_coord_diff(x16, row, col):
    mesh = plsc.VectorSubcoreMesh(core_axis_name="c", subcore_axis_name="s")
    CH = 200
    n_chunks = _EPW // CH

    @functools.partial(
        pl.kernel, mesh=mesh,
        out_type=jax.ShapeDtypeStruct((E, XW), _f32),
        compiler_params=pltpu.CompilerParams(use_tc_tiling_on_sc=False),
        scratch_types=[
            pltpu.VMEM((CH,), jnp.int32),
            pltpu.VMEM((CH,), jnp.int32),
            pltpu.VMEM((CH, XW), _f32),
            pltpu.VMEM((CH, XW), _f32),
            pltpu.SemaphoreType.DMA,
            pltpu.SemaphoreType.DMA,
        ],
    )
    def k(x_hbm, row_hbm, col_hbm, diff_hbm,
          idxr, idxc, bufXa, bufXb, semXa, semXb):
        wid = lax.axis_index("s") * _NC + lax.axis_index("c")
        base0 = wid * _EPW

        def chunk(i, carry):
            base = base0 + i * CH
            pltpu.sync_copy(row_hbm.at[pl.ds(base, CH)], idxr)
            pltpu.sync_copy(col_hbm.at[pl.ds(base, CH)], idxc)
            cpXa = pltpu.async_copy(x_hbm.at[idxr], bufXa, semXa)
            cpXb = pltpu.async_copy(x_hbm.at[idxc], bufXb, semXb)
            cpXa.wait()
            cpXb.wait()

            def sub_row(r, c2):
                sl = pl.ds(0, 16)
                bufXa[r, sl] = bufXa[r, sl] - bufXb[r, sl]
                return c2

            lax.fori_loop(0, CH, sub_row, 0)
            pltpu.sync_copy(bufXa, diff_hbm.at[pl.ds(base, CH)])
            return carry

        lax.fori_loop(0, n_chunks, chunk, 0)

    return k(x16, row, col)


_NPAD = 10240
_NPS_P = _NPAD // _NS


def _sc_segment_sum(vals, row, F, CH):
    mesh = plsc.VectorSubcoreMesh(core_axis_name="c", subcore_axis_name="s")
    n_chunks = _EPW // CH
    ZR = 64
    params = (pltpu.CompilerParams(use_tc_tiling_on_sc=False)
              if F != H else None)

    @functools.partial(
        pl.kernel, mesh=mesh,
        compiler_params=params,
        out_type=jax.ShapeDtypeStruct((_NC, _NPAD, F), _f32),
        scratch_types=[
            pltpu.VMEM_SHARED((_NPAD, F), _f32),
            pltpu.VMEM((CH,), jnp.int32),
            pltpu.VMEM((CH, F), _f32),
            pltpu.VMEM((ZR, F), _f32),
        ],
    )
    def k(vals_hbm, row_hbm, out_hbm, acc, idxv, buf, bufZ):
        c = lax.axis_index("c")
        s = lax.axis_index("s")
        wid = s * _NC + c

        def zrow(r, c2):
            for kk in range(F // 16):
                bufZ[r, pl.ds(kk * 16, 16)] = jnp.zeros((16,), _f32)
            return c2

        lax.fori_loop(0, ZR, zrow, 0)

        def zcopy(m, c2):
            pltpu.sync_copy(bufZ, acc.at[pl.ds(s * _NPS_P + m * ZR, ZR)])
            return c2

        lax.fori_loop(0, _NPS_P // ZR, zcopy, 0)
        plsc.subcore_barrier()

        def chunk(i, c2):
            base = wid * _EPW + i * CH
            pltpu.sync_copy(row_hbm.at[pl.ds(base, CH)], idxv)
            pltpu.sync_copy(vals_hbm.at[pl.ds(base, CH)], buf)
            pltpu.sync_copy(buf, acc.at[idxv], add=True)
            return c2

        lax.fori_loop(0, n_chunks, chunk, 0)
        plsc.subcore_barrier()
        pltpu.sync_copy(acc.at[pl.ds(s * _NPS_P, _NPS_P)],
                        out_hbm.at[c, pl.ds(s * _NPS_P, _NPS_P)])

    return k(vals, row)


_BN = 1000
_BE = 2000


def _tc_proj_pair(h, Wa, Wb):
    def body(h_ref, wa_ref, wb_ref, a_ref, b_ref):
        hb = h_ref[...]
        a_ref[...] = jnp.dot(hb, wa_ref[...], preferred_element_type=_f32)
        b_ref[...] = jnp.dot(hb, wb_ref[...], preferred_element_type=_f32)

    grid = N // _BN
    return pl.pallas_call(
        body,
        grid=(grid,),
        in_specs=[
            pl.BlockSpec((_BN, H), lambda i: (i, 0)),
            pl.BlockSpec((H, H), lambda i: (0, 0)),
            pl.BlockSpec((H, H), lambda i: (0, 0)),
        ],
        out_specs=[
            pl.BlockSpec((_BN, H), lambda i: (i, 0)),
            pl.BlockSpec((_BN, H), lambda i: (i, 0)),
        ],
        out_shape=[
            jax.ShapeDtypeStruct((N, H), _f32),
            jax.ShapeDtypeStruct((N, H), _f32),
        ],
    )(h, Wa, Wb)


def _tc_edge_mlp(pre, diff, eattr, w_r, w_e, eb1, eW2, eb2, aWt, ab):
    def body(pre_ref, d_ref, ea_ref, wr_ref, we_ref, b1_ref, w2_ref, b2_ref,
             awt_ref, ab_ref, ef_ref):
        d = d_ref[...]
        radial = jnp.sum(d * d, axis=1, keepdims=True)
        z1 = (pre_ref[...] + radial * wr_ref[...] + ea_ref[...] * we_ref[...]
              + b1_ref[...])
        u = _silu(z1)
        z2 = jnp.dot(u, w2_ref[...], preferred_element_type=_f32) + b2_ref[...]
        mij = _silu(z2)
        att = jax.nn.sigmoid(
            jnp.sum(mij * awt_ref[...], axis=1, keepdims=True) + ab_ref[...])
        ef_ref[...] = mij * att

    grid = E // _BE
    return pl.pallas_call(
        body,
        grid=(grid,),
        in_specs=[
            pl.BlockSpec((_BE, H), lambda i: (i, 0)),
            pl.BlockSpec((_BE, XW), lambda i: (i, 0)),
            pl.BlockSpec((_BE, 1), lambda i: (i, 0)),
            pl.BlockSpec((1, H), lambda i: (0, 0)),
            pl.BlockSpec((1, H), lambda i: (0, 0)),
            pl.BlockSpec((1, H), lambda i: (0, 0)),
            pl.BlockSpec((H, H), lambda i: (0, 0)),
            pl.BlockSpec((1, H), lambda i: (0, 0)),
            pl.BlockSpec((1, H), lambda i: (0, 0)),
            pl.BlockSpec((1, 1), lambda i: (0, 0)),
        ],
        out_specs=pl.BlockSpec((_BE, H), lambda i: (i, 0)),
        out_shape=jax.ShapeDtypeStruct((E, H), _f32),
    )(pre, diff, eattr, w_r, w_e, eb1, eW2, eb2, aWt, ab)


def _tc_coord_mlp(pre, diff, eattr, w_r, w_e, eb1, eW2, eb2, w3t):
    def body(pre_ref, d_ref, ea_ref, wr_ref, we_ref, b1_ref, w2_ref, b2_ref,
             w3t_ref, tr_ref):
        d = d_ref[...]
        radial = jnp.sum(d * d, axis=1, keepdims=True)
        z1 = (pre_ref[...] + radial * wr_ref[...] + ea_ref[...] * we_ref[...]
              + b1_ref[...])
        u = _silu(z1)
        z2 = jnp.dot(u, w2_ref[...], preferred_element_type=_f32) + b2_ref[...]
        mij = _silu(z2)
        t = jnp.sum(mij * w3t_ref[...], axis=1, keepdims=True)
        cd = d / (jnp.sqrt(radial + 1e-8) + 1.0)
        tr_ref[...] = cd * t

    grid = E // _BE
    return pl.pallas_call(
        body,
        grid=(grid,),
        in_specs=[
            pl.BlockSpec((_BE, H), lambda i: (i, 0)),
            pl.BlockSpec((_BE, XW), lambda i: (i, 0)),
            pl.BlockSpec((_BE, 1), lambda i: (i, 0)),
            pl.BlockSpec((1, H), lambda i: (0, 0)),
            pl.BlockSpec((1, H), lambda i: (0, 0)),
            pl.BlockSpec((1, H), lambda i: (0, 0)),
            pl.BlockSpec((H, H), lambda i: (0, 0)),
            pl.BlockSpec((1, H), lambda i: (0, 0)),
            pl.BlockSpec((1, H), lambda i: (0, 0)),
        ],
        out_specs=pl.BlockSpec((_BE, XW), lambda i: (i, 0)),
        out_shape=jax.ShapeDtypeStruct((E, XW), _f32),
    )(pre, diff, eattr, w_r, w_e, eb1, eW2, eb2, w3t)


def _tc_node_mlp(h, aggp, nW1, nb1, nW2, nb2, Wa_next, Wb_next):
    def body(h_ref, p_ref, w1_ref, b1_ref, w2_ref, b2_ref, wa_ref, wb_ref,
             hn_ref, a_ref, b_ref):
        hb = h_ref[...]
        agg = (p_ref[0] + p_ref[1]) * 0.01
        w1 = w1_ref[...]
        z = (jnp.dot(hb, w1[:H], preferred_element_type=_f32)
             + jnp.dot(agg, w1[H:], preferred_element_type=_f32)
             + b1_ref[...])
        u = _silu(z)
        hn = hb + jnp.dot(u, w2_ref[...], preferred_element_type=_f32) + b2_ref[...]
        hn_ref[...] = hn
        a_ref[...] = jnp.dot(hn, wa_ref[...], preferred_element_type=_f32)
        b_ref[...] = jnp.dot(hn, wb_ref[...], preferred_element_type=_f32)

    grid = N // _BN
    return pl.pallas_call(
        body,
        grid=(grid,),
        in_specs=[
            pl.BlockSpec((_BN, H), lambda i: (i, 0)),
            pl.BlockSpec((2, _BN, H), lambda i: (0, i, 0)),
            pl.BlockSpec((2 * H, H), lambda i: (0, 0)),
            pl.BlockSpec((1, H), lambda i: (0, 0)),
            pl.BlockSpec((H, H), lambda i: (0, 0)),
            pl.BlockSpec((1, H), lambda i: (0, 0)),
            pl.BlockSpec((H, H), lambda i: (0, 0)),
            pl.BlockSpec((H, H), lambda i: (0, 0)),
        ],
        out_specs=[
            pl.BlockSpec((_BN, H), lambda i: (i, 0)),
            pl.BlockSpec((_BN, H), lambda i: (i, 0)),
            pl.BlockSpec((_BN, H), lambda i: (i, 0)),
        ],
        out_shape=[
            jax.ShapeDtypeStruct((N, H), _f32),
            jax.ShapeDtypeStruct((N, H), _f32),
            jax.ShapeDtypeStruct((N, H), _f32),
        ],
    )(h, aggp, nW1, nb1, nW2, nb2, Wa_next, Wb_next)


def _tc_coord_update(x16, xaggp):
    def body(x_ref, p_ref, o_ref):
        o_ref[...] = x_ref[...] + (p_ref[0] + p_ref[1]) * 0.01

    grid = N // _BN
    return pl.pallas_call(
        body,
        grid=(grid,),
        in_specs=[
            pl.BlockSpec((_BN, XW), lambda i: (i, 0)),
            pl.BlockSpec((2, _BN, XW), lambda i: (0, i, 0)),
        ],
        out_specs=pl.BlockSpec((_BN, XW), lambda i: (i, 0)),
        out_shape=jax.ShapeDtypeStruct((N, XW), _f32),
    )(x16, xaggp)


def _gather_sum(A, B, row, col):
    return _sc_gather_sum(A, B, row, col)


def _coord_diff(x16, row, col):
    return _sc_coord_diff(x16, row, col)


def _segment_sum(vals, row, F, CH):
    return _sc_segment_sum(vals, row, F, CH)


def kernel(h, x, edge_index, edge_attr,
           gcl0_eW1, gcl0_eb1, gcl0_eW2, gcl0_eb2, gcl0_nW1, gcl0_nb1,
           gcl0_nW2, gcl0_nb2, gcl0_aW, gcl0_ab,
           gcl1_eW1, gcl1_eb1, gcl1_eW2, gcl1_eb2, gcl1_nW1, gcl1_nb1,
           gcl1_nW2, gcl1_nb2, gcl1_aW, gcl1_ab,
           eq_W1, eq_b1, eq_W2, eq_b2, eq_W3):
    row = edge_index[0]
    col = edge_index[1]
    x16 = jnp.pad(x, ((0, 0), (0, XW - 3)))
    ea = edge_attr

    def esplit(W1):
        return (W1[:H], W1[H:2 * H], W1[2 * H:2 * H + 1].reshape(1, H),
                W1[2 * H + 1:].reshape(1, H))

    Wa0, Wb0, wr0, we0 = esplit(gcl0_eW1)
    A0, B0 = _tc_proj_pair(h, Wa0, Wb0)
    diff = _coord_diff(x16, row, col)
    pre0 = _gather_sum(A0, B0, row, col)
    ef0 = _tc_edge_mlp(pre0, diff, ea, wr0, we0, gcl0_eb1.reshape(1, H),
                       gcl0_eW2, gcl0_eb2.reshape(1, H),
                       gcl0_aW.reshape(1, H), gcl0_ab.reshape(1, 1))
    agg0 = _segment_sum(ef0, row, H, 200)
    Wa1, Wb1, wr1, we1 = esplit(gcl1_eW1)
    h1, A1, B1 = _tc_node_mlp(h, agg0, gcl0_nW1, gcl0_nb1.reshape(1, H),
                              gcl0_nW2, gcl0_nb2.reshape(1, H), Wa1, Wb1)

    pre1 = _gather_sum(A1, B1, row, col)
    ef1 = _tc_edge_mlp(pre1, diff, ea, wr1, we1, gcl1_eb1.reshape(1, H),
                       gcl1_eW2, gcl1_eb2.reshape(1, H),
                       gcl1_aW.reshape(1, H), gcl1_ab.reshape(1, 1))
    agg1 = _segment_sum(ef1, row, H, 200)
    Wa2, Wb2, wr2, we2 = esplit(eq_W1)
    h2, A2, B2 = _tc_node_mlp(h1, agg1, gcl1_nW1, gcl1_nb1.reshape(1, H),
                              gcl1_nW2, gcl1_nb2.reshape(1, H), Wa2, Wb2)

    pre2 = _gather_sum(A2, B2, row, col)
    trans = _tc_coord_mlp(pre2, diff, ea, wr2, we2, eq_b1.reshape(1, H),
                          eq_W2, eq_b2.reshape(1, H), eq_W3.reshape(1, H))
    xagg = _segment_sum(trans, row, XW, 200)
    x16_out = _tc_coord_update(x16, xagg)

    return (h2, x16_out[:, :3])


# speedup vs baseline: 1.1515x; 1.1515x over previous
import functools

import jax
import jax.numpy as jnp
from jax import lax
from jax.experimental import pallas as pl
from jax.experimental.pallas import tpu as pltpu
from jax.experimental.pallas import tpu_sc as plsc

N = 10000
E = 320000
H = 128
XW = 16

_NC = 2
_NS = 16
_NW = _NC * _NS
_EPW = E // _NW
_NPS = N // _NS

_f32 = jnp.float32


def _silu(v):
    return v * jax.nn.sigmoid(v)


def _sc_gather_sum(A, B, row, col):
    ne = row.shape[0]
    epw = ne // _NW
    CH = 200
    n_chunks = epw // CH

    mesh = plsc.VectorSubcoreMesh(core_axis_name="c", subcore_axis_name="s")

    @functools.partial(
        pl.kernel, mesh=mesh,
        out_type=jax.ShapeDtypeStruct((ne, H), _f32),
        scratch_types=[
            pltpu.VMEM((CH,), jnp.int32),
            pltpu.VMEM((CH,), jnp.int32),
            pltpu.VMEM((CH, H), _f32),
            pltpu.VMEM((CH, H), _f32),
            pltpu.SemaphoreType.DMA,
            pltpu.SemaphoreType.DMA,
        ],
    )
    def k(A_hbm, B_hbm, row_hbm, col_hbm, pre_hbm,
          idxr, idxc, bufA, bufB, semA, semB):
        wid = lax.axis_index("s") * _NC + lax.axis_index("c")
        base0 = wid * epw

        def chunk(i, carry):
            base = base0 + i * CH
            pltpu.sync_copy(row_hbm.at[pl.ds(base, CH)], idxr)
            pltpu.sync_copy(col_hbm.at[pl.ds(base, CH)], idxc)
            cpA = pltpu.async_copy(A_hbm.at[idxr], bufA, semA)
            cpB = pltpu.async_copy(B_hbm.at[idxc], bufB, semB)
            cpA.wait()
            cpB.wait()

            def add_row(r, c2):
                for kk in range(H // 16):
                    sl = pl.ds(kk * 16, 16)
                    bufA[r, sl] = bufA[r, sl] + bufB[r, sl]
                return c2

            lax.fori_loop(0, CH, add_row, 0)
            pltpu.sync_copy(bufA, pre_hbm.at[pl.ds(base, CH)])
            return carry

        lax.fori_loop(0, n_chunks, chunk, 0)

    return k(A, B, row, col)


def _sc_coord_diff(x16, row, col):
    mesh = plsc.VectorSubcoreMesh(core_axis_name="c", subcore_axis_name="s")
    CH = 200
    n_chunks = _EPW // CH

    @functools.partial(
        pl.kernel, mesh=mesh,
        out_type=jax.ShapeDtypeStruct((E, XW), _f32),
        compiler_params=pltpu.CompilerParams(use_tc_tiling_on_sc=False),
        scratch_types=[
            pltpu.VMEM((CH,), jnp.int32),
            pltpu.VMEM((CH,), jnp.int32),
            pltpu.VMEM((CH, XW), _f32),
            pltpu.VMEM((CH, XW), _f32),
            pltpu.SemaphoreType.DMA,
            pltpu.SemaphoreType.DMA,
        ],
    )
    def k(x_hbm, row_hbm, col_hbm, diff_hbm,
          idxr, idxc, bufXa, bufXb, semXa, semXb):
        wid = lax.axis_index("s") * _NC + lax.axis_index("c")
        base0 = wid * _EPW

        def chunk(i, carry):
            base = base0 + i * CH
            pltpu.sync_copy(row_hbm.at[pl.ds(base, CH)], idxr)
            pltpu.sync_copy(col_hbm.at[pl.ds(base, CH)], idxc)
            cpXa = pltpu.async_copy(x_hbm.at[idxr], bufXa, semXa)
            cpXb = pltpu.async_copy(x_hbm.at[idxc], bufXb, semXb)
            cpXa.wait()
            cpXb.wait()

            def sub_row(r, c2):
                sl = pl.ds(0, 16)
                bufXa[r, sl] = bufXa[r, sl] - bufXb[r, sl]
                return c2

            lax.fori_loop(0, CH, sub_row, 0)
            pltpu.sync_copy(bufXa, diff_hbm.at[pl.ds(base, CH)])
            return carry

        lax.fori_loop(0, n_chunks, chunk, 0)

    return k(x16, row, col)


_NPAD = 10240
_NPS_P = _NPAD // _NS


def _sc_segment_sum(vals, row, F, CH):
    ne = vals.shape[0]
    epw = ne // _NW
    mesh = plsc.VectorSubcoreMesh(core_axis_name="c", subcore_axis_name="s")
    n_chunks = epw // CH
    ZR = 64
    params = (pltpu.CompilerParams(use_tc_tiling_on_sc=False)
              if F != H else None)

    @functools.partial(
        pl.kernel, mesh=mesh,
        compiler_params=params,
        out_type=jax.ShapeDtypeStruct((_NC, _NPAD, F), _f32),
        scratch_types=[
            pltpu.VMEM_SHARED((_NPAD, F), _f32),
            pltpu.VMEM((CH,), jnp.int32),
            pltpu.VMEM((CH, F), _f32),
            pltpu.VMEM((ZR, F), _f32),
        ],
    )
    def k(vals_hbm, row_hbm, out_hbm, acc, idxv, buf, bufZ):
        c = lax.axis_index("c")
        s = lax.axis_index("s")
        wid = s * _NC + c

        def zrow(r, c2):
            for kk in range(F // 16):
                bufZ[r, pl.ds(kk * 16, 16)] = jnp.zeros((16,), _f32)
            return c2

        lax.fori_loop(0, ZR, zrow, 0)

        def zcopy(m, c2):
            pltpu.sync_copy(bufZ, acc.at[pl.ds(s * _NPS_P + m * ZR, ZR)])
            return c2

        lax.fori_loop(0, _NPS_P // ZR, zcopy, 0)
        plsc.subcore_barrier()

        def chunk(i, c2):
            base = wid * epw + i * CH
            pltpu.sync_copy(row_hbm.at[pl.ds(base, CH)], idxv)
            pltpu.sync_copy(vals_hbm.at[pl.ds(base, CH)], buf)
            pltpu.sync_copy(buf, acc.at[idxv], add=True)
            return c2

        lax.fori_loop(0, n_chunks, chunk, 0)
        plsc.subcore_barrier()
        pltpu.sync_copy(acc.at[pl.ds(s * _NPS_P, _NPS_P)],
                        out_hbm.at[c, pl.ds(s * _NPS_P, _NPS_P)])

    return k(vals, row)


_BN = 1000
_BE = 2000


def _tc_proj_pair(h, Wa, Wb):
    def body(h_ref, wa_ref, wb_ref, a_ref, b_ref):
        hb = h_ref[...]
        a_ref[...] = jnp.dot(hb, wa_ref[...], preferred_element_type=_f32)
        b_ref[...] = jnp.dot(hb, wb_ref[...], preferred_element_type=_f32)

    grid = N // _BN
    return pl.pallas_call(
        body,
        grid=(grid,),
        in_specs=[
            pl.BlockSpec((_BN, H), lambda i: (i, 0)),
            pl.BlockSpec((H, H), lambda i: (0, 0)),
            pl.BlockSpec((H, H), lambda i: (0, 0)),
        ],
        out_specs=[
            pl.BlockSpec((_BN, H), lambda i: (i, 0)),
            pl.BlockSpec((_BN, H), lambda i: (i, 0)),
        ],
        out_shape=[
            jax.ShapeDtypeStruct((N, H), _f32),
            jax.ShapeDtypeStruct((N, H), _f32),
        ],
    )(h, Wa, Wb)


def _tc_edge_mlp(pre, diff, eattr, w_r, w_e, eb1, eW2, eb2, aWt, ab, off=0):
    ne = pre.shape[0]
    ob = off // _BE
    def body(pre_ref, d_ref, ea_ref, wr_ref, we_ref, b1_ref, w2_ref, b2_ref,
             awt_ref, ab_ref, ef_ref):
        d = d_ref[...]
        radial = jnp.sum(d * d, axis=1, keepdims=True)
        z1 = (pre_ref[...] + radial * wr_ref[...] + ea_ref[...] * we_ref[...]
              + b1_ref[...])
        u = _silu(z1)
        z2 = jnp.dot(u, w2_ref[...], preferred_element_type=_f32) + b2_ref[...]
        mij = _silu(z2)
        att = jax.nn.sigmoid(
            jnp.sum(mij * awt_ref[...], axis=1, keepdims=True) + ab_ref[...])
        ef_ref[...] = mij * att

    grid = ne // _BE
    return pl.pallas_call(
        body,
        grid=(grid,),
        in_specs=[
            pl.BlockSpec((_BE, H), lambda i: (i, 0)),
            pl.BlockSpec((_BE, XW), lambda i: (i + ob, 0)),
            pl.BlockSpec((_BE, 1), lambda i: (i + ob, 0)),
            pl.BlockSpec((1, H), lambda i: (0, 0)),
            pl.BlockSpec((1, H), lambda i: (0, 0)),
            pl.BlockSpec((1, H), lambda i: (0, 0)),
            pl.BlockSpec((H, H), lambda i: (0, 0)),
            pl.BlockSpec((1, H), lambda i: (0, 0)),
            pl.BlockSpec((1, H), lambda i: (0, 0)),
            pl.BlockSpec((1, 1), lambda i: (0, 0)),
        ],
        out_specs=pl.BlockSpec((_BE, H), lambda i: (i, 0)),
        out_shape=jax.ShapeDtypeStruct((ne, H), _f32),
    )(pre, diff, eattr, w_r, w_e, eb1, eW2, eb2, aWt, ab)


def _tc_coord_mlp(pre, diff, eattr, w_r, w_e, eb1, eW2, eb2, w3t, off=0):
    ne = pre.shape[0]
    ob = off // _BE
    def body(pre_ref, d_ref, ea_ref, wr_ref, we_ref, b1_ref, w2_ref, b2_ref,
             w3t_ref, tr_ref):
        d = d_ref[...]
        radial = jnp.sum(d * d, axis=1, keepdims=True)
        z1 = (pre_ref[...] + radial * wr_ref[...] + ea_ref[...] * we_ref[...]
              + b1_ref[...])
        u = _silu(z1)
        z2 = jnp.dot(u, w2_ref[...], preferred_element_type=_f32) + b2_ref[...]
        mij = _silu(z2)
        t = jnp.sum(mij * w3t_ref[...], axis=1, keepdims=True)
        cd = d / (jnp.sqrt(radial + 1e-8) + 1.0)
        tr_ref[...] = cd * t

    grid = ne // _BE
    return pl.pallas_call(
        body,
        grid=(grid,),
        in_specs=[
            pl.BlockSpec((_BE, H), lambda i: (i, 0)),
            pl.BlockSpec((_BE, XW), lambda i: (i + ob, 0)),
            pl.BlockSpec((_BE, 1), lambda i: (i + ob, 0)),
            pl.BlockSpec((1, H), lambda i: (0, 0)),
            pl.BlockSpec((1, H), lambda i: (0, 0)),
            pl.BlockSpec((1, H), lambda i: (0, 0)),
            pl.BlockSpec((H, H), lambda i: (0, 0)),
            pl.BlockSpec((1, H), lambda i: (0, 0)),
            pl.BlockSpec((1, H), lambda i: (0, 0)),
        ],
        out_specs=pl.BlockSpec((_BE, XW), lambda i: (i, 0)),
        out_shape=jax.ShapeDtypeStruct((ne, XW), _f32),
    )(pre, diff, eattr, w_r, w_e, eb1, eW2, eb2, w3t)


def _tc_node_mlp(h, aggp, nW1, nb1, nW2, nb2, Wa_next, Wb_next):
    def body(h_ref, p_ref, w1_ref, b1_ref, w2_ref, b2_ref, wa_ref, wb_ref,
             hn_ref, a_ref, b_ref):
        hb = h_ref[...]
        agg = (p_ref[0] + p_ref[1] + p_ref[2] + p_ref[3]) * 0.01
        w1 = w1_ref[...]
        z = (jnp.dot(hb, w1[:H], preferred_element_type=_f32)
             + jnp.dot(agg, w1[H:], preferred_element_type=_f32)
             + b1_ref[...])
        u = _silu(z)
        hn = hb + jnp.dot(u, w2_ref[...], preferred_element_type=_f32) + b2_ref[...]
        hn_ref[...] = hn
        a_ref[...] = jnp.dot(hn, wa_ref[...], preferred_element_type=_f32)
        b_ref[...] = jnp.dot(hn, wb_ref[...], preferred_element_type=_f32)

    grid = N // _BN
    return pl.pallas_call(
        body,
        grid=(grid,),
        in_specs=[
            pl.BlockSpec((_BN, H), lambda i: (i, 0)),
            pl.BlockSpec((4, _BN, H), lambda i: (0, i, 0)),
            pl.BlockSpec((2 * H, H), lambda i: (0, 0)),
            pl.BlockSpec((1, H), lambda i: (0, 0)),
            pl.BlockSpec((H, H), lambda i: (0, 0)),
            pl.BlockSpec((1, H), lambda i: (0, 0)),
            pl.BlockSpec((H, H), lambda i: (0, 0)),
            pl.BlockSpec((H, H), lambda i: (0, 0)),
        ],
        out_specs=[
            pl.BlockSpec((_BN, H), lambda i: (i, 0)),
            pl.BlockSpec((_BN, H), lambda i: (i, 0)),
            pl.BlockSpec((_BN, H), lambda i: (i, 0)),
        ],
        out_shape=[
            jax.ShapeDtypeStruct((N, H), _f32),
            jax.ShapeDtypeStruct((N, H), _f32),
            jax.ShapeDtypeStruct((N, H), _f32),
        ],
    )(h, aggp, nW1, nb1, nW2, nb2, Wa_next, Wb_next)


def _tc_coord_update(x16, xaggp):
    def body(x_ref, p_ref, o_ref):
        o_ref[...] = x_ref[...] + (p_ref[0] + p_ref[1] + p_ref[2] + p_ref[3]) * 0.01

    grid = N // _BN
    return pl.pallas_call(
        body,
        grid=(grid,),
        in_specs=[
            pl.BlockSpec((_BN, XW), lambda i: (i, 0)),
            pl.BlockSpec((4, _BN, XW), lambda i: (0, i, 0)),
        ],
        out_specs=pl.BlockSpec((_BN, XW), lambda i: (i, 0)),
        out_shape=jax.ShapeDtypeStruct((N, XW), _f32),
    )(x16, xaggp)


_NSPLIT = 2
_ES = E // _NSPLIT


def _edge_stage(A, B, row, col, diff, ea, wr, we, eb1, eW2, eb2, aWt, ab):
    parts = []
    for s in range(_NSPLIT):
        off = s * _ES
        rs = row[off:off + _ES]
        cs = col[off:off + _ES]
        pre = _sc_gather_sum(A, B, rs, cs)
        ef = _tc_edge_mlp(pre, diff, ea, wr, we, eb1, eW2, eb2, aWt, ab,
                          off=off)
        parts.append(_sc_segment_sum(ef, rs, H, 200))
    return jnp.concatenate(parts, axis=0)


def _coord_stage(A, B, row, col, diff, ea, wr, we, eb1, eW2, eb2, w3t):
    parts = []
    for s in range(_NSPLIT):
        off = s * _ES
        rs = row[off:off + _ES]
        cs = col[off:off + _ES]
        pre = _sc_gather_sum(A, B, rs, cs)
        trans = _tc_coord_mlp(pre, diff, ea, wr, we, eb1, eW2, eb2, w3t,
                              off=off)
        parts.append(_sc_segment_sum(trans, rs, XW, 200))
    return jnp.concatenate(parts, axis=0)


def kernel(h, x, edge_index, edge_attr,
           gcl0_eW1, gcl0_eb1, gcl0_eW2, gcl0_eb2, gcl0_nW1, gcl0_nb1,
           gcl0_nW2, gcl0_nb2, gcl0_aW, gcl0_ab,
           gcl1_eW1, gcl1_eb1, gcl1_eW2, gcl1_eb2, gcl1_nW1, gcl1_nb1,
           gcl1_nW2, gcl1_nb2, gcl1_aW, gcl1_ab,
           eq_W1, eq_b1, eq_W2, eq_b2, eq_W3):
    row = edge_index[0]
    col = edge_index[1]
    x16 = jnp.pad(x, ((0, 0), (0, XW - 3)))
    ea = edge_attr

    def esplit(W1):
        return (W1[:H], W1[H:2 * H], W1[2 * H:2 * H + 1].reshape(1, H),
                W1[2 * H + 1:].reshape(1, H))

    Wa0, Wb0, wr0, we0 = esplit(gcl0_eW1)
    A0, B0 = _tc_proj_pair(h, Wa0, Wb0)
    diff = _sc_coord_diff(x16, row, col)
    agg0 = _edge_stage(A0, B0, row, col, diff, ea, wr0, we0,
                       gcl0_eb1.reshape(1, H), gcl0_eW2,
                       gcl0_eb2.reshape(1, H), gcl0_aW.reshape(1, H),
                       gcl0_ab.reshape(1, 1))
    Wa1, Wb1, wr1, we1 = esplit(gcl1_eW1)
    h1, A1, B1 = _tc_node_mlp(h, agg0, gcl0_nW1, gcl0_nb1.reshape(1, H),
                              gcl0_nW2, gcl0_nb2.reshape(1, H), Wa1, Wb1)

    agg1 = _edge_stage(A1, B1, row, col, diff, ea, wr1, we1,
                       gcl1_eb1.reshape(1, H), gcl1_eW2,
                       gcl1_eb2.reshape(1, H), gcl1_aW.reshape(1, H),
                       gcl1_ab.reshape(1, 1))
    Wa2, Wb2, wr2, we2 = esplit(eq_W1)
    h2, A2, B2 = _tc_node_mlp(h1, agg1, gcl1_nW1, gcl1_nb1.reshape(1, H),
                              gcl1_nW2, gcl1_nb2.reshape(1, H), Wa2, Wb2)

    xagg = _coord_stage(A2, B2, row, col, diff, ea, wr2, we2,
                        eq_b1.reshape(1, H), eq_W2, eq_b2.reshape(1, H),
                        eq_W3.reshape(1, H))
    x16_out = _tc_coord_update(x16, xagg)

    return (h2, x16_out[:, :3])


# speedup vs baseline: 1.1643x; 1.0111x over previous
import functools

import jax
import jax.numpy as jnp
from jax import lax
from jax.experimental import pallas as pl
from jax.experimental.pallas import tpu as pltpu
from jax.experimental.pallas import tpu_sc as plsc

N = 10000
E = 320000
H = 128
XW = 16

_NC = 2
_NS = 16
_NW = _NC * _NS
_EPW = E // _NW
_NPS = N // _NS

_f32 = jnp.float32


def _silu(v):
    return v * jax.nn.sigmoid(v)


def _sc_gather_sum(A, B, row, col):
    ne = row.shape[0]
    epw = ne // _NW
    CH = 200
    n_chunks = epw // CH

    mesh = plsc.VectorSubcoreMesh(core_axis_name="c", subcore_axis_name="s")
    ident = jnp.arange(CH, dtype=jnp.int32)

    @functools.partial(
        pl.kernel, mesh=mesh,
        out_type=jax.ShapeDtypeStruct((ne, H), _f32),
        scratch_types=[
            pltpu.VMEM_SHARED((_NS * CH, H), _f32),
            pltpu.VMEM((CH,), jnp.int32),
            pltpu.VMEM((CH,), jnp.int32),
            pltpu.VMEM((CH,), jnp.int32),
            pltpu.VMEM((CH, H), _f32),
            pltpu.VMEM((CH, H), _f32),
            pltpu.SemaphoreType.DMA,
            pltpu.SemaphoreType.DMA,
        ],
    )
    def k(A_hbm, B_hbm, row_hbm, col_hbm, ident_hbm, pre_hbm,
          acc, idxr, idxc, idxi, bufA, bufB, semA, semB):
        s = lax.axis_index("s")
        wid = s * _NC + lax.axis_index("c")
        base0 = wid * epw
        stripe = s * CH
        pltpu.sync_copy(ident_hbm, idxi)

        def ioff(kk, c2):
            sl = pl.ds(kk * 16, 16)
            idxi[sl] = idxi[sl] + stripe
            return c2

        lax.fori_loop(0, CH // 16, ioff, 0)

        def chunk(i, carry):
            base = base0 + i * CH
            pltpu.sync_copy(row_hbm.at[pl.ds(base, CH)], idxr)
            pltpu.sync_copy(col_hbm.at[pl.ds(base, CH)], idxc)
            cpA = pltpu.async_copy(A_hbm.at[idxr], bufA, semA)
            cpB = pltpu.async_copy(B_hbm.at[idxc], bufB, semB)
            cpA.wait()
            pltpu.sync_copy(bufA, acc.at[pl.ds(stripe, CH)])
            cpB.wait()
            pltpu.sync_copy(bufB, acc.at[idxi], add=True)
            pltpu.sync_copy(acc.at[pl.ds(stripe, CH)],
                            pre_hbm.at[pl.ds(base, CH)])
            return carry

        lax.fori_loop(0, n_chunks, chunk, 0)

    return k(A, B, row, col, ident)


def _sc_coord_diff(x16, row, col):
    mesh = plsc.VectorSubcoreMesh(core_axis_name="c", subcore_axis_name="s")
    CH = 200
    n_chunks = _EPW // CH

    @functools.partial(
        pl.kernel, mesh=mesh,
        out_type=jax.ShapeDtypeStruct((E, XW), _f32),
        compiler_params=pltpu.CompilerParams(use_tc_tiling_on_sc=False),
        scratch_types=[
            pltpu.VMEM((CH,), jnp.int32),
            pltpu.VMEM((CH,), jnp.int32),
            pltpu.VMEM((CH, XW), _f32),
            pltpu.VMEM((CH, XW), _f32),
            pltpu.SemaphoreType.DMA,
            pltpu.SemaphoreType.DMA,
        ],
    )
    def k(x_hbm, row_hbm, col_hbm, diff_hbm,
          idxr, idxc, bufXa, bufXb, semXa, semXb):
        wid = lax.axis_index("s") * _NC + lax.axis_index("c")
        base0 = wid * _EPW

        def chunk(i, carry):
            base = base0 + i * CH
            pltpu.sync_copy(row_hbm.at[pl.ds(base, CH)], idxr)
            pltpu.sync_copy(col_hbm.at[pl.ds(base, CH)], idxc)
            cpXa = pltpu.async_copy(x_hbm.at[idxr], bufXa, semXa)
            cpXb = pltpu.async_copy(x_hbm.at[idxc], bufXb, semXb)
            cpXa.wait()
            cpXb.wait()

            def sub_row(r, c2):
                sl = pl.ds(0, 16)
                bufXa[r, sl] = bufXa[r, sl] - bufXb[r, sl]
                return c2

            lax.fori_loop(0, CH, sub_row, 0)
            pltpu.sync_copy(bufXa, diff_hbm.at[pl.ds(base, CH)])
            return carry

        lax.fori_loop(0, n_chunks, chunk, 0)

    return k(x16, row, col)


_NPAD = 10240
_NPS_P = _NPAD // _NS


def _sc_segment_sum(vals, row, F, CH):
    ne = vals.shape[0]
    epw = ne // _NW
    mesh = plsc.VectorSubcoreMesh(core_axis_name="c", subcore_axis_name="s")
    n_chunks = epw // CH
    ZR = 64
    params = (pltpu.CompilerParams(use_tc_tiling_on_sc=False)
              if F != H else None)

    @functools.partial(
        pl.kernel, mesh=mesh,
        compiler_params=params,
        out_type=jax.ShapeDtypeStruct((_NC, _NPAD, F), _f32),
        scratch_types=[
            pltpu.VMEM_SHARED((_NPAD, F), _f32),
            pltpu.VMEM((CH,), jnp.int32),
            pltpu.VMEM((CH, F), _f32),
            pltpu.VMEM((ZR, F), _f32),
        ],
    )
    def k(vals_hbm, row_hbm, out_hbm, acc, idxv, buf, bufZ):
        c = lax.axis_index("c")
        s = lax.axis_index("s")
        wid = s * _NC + c

        def zrow(r, c2):
            for kk in range(F // 16):
                bufZ[r, pl.ds(kk * 16, 16)] = jnp.zeros((16,), _f32)
            return c2

        lax.fori_loop(0, ZR, zrow, 0)

        def zcopy(m, c2):
            pltpu.sync_copy(bufZ, acc.at[pl.ds(s * _NPS_P + m * ZR, ZR)])
            return c2

        lax.fori_loop(0, _NPS_P // ZR, zcopy, 0)
        plsc.subcore_barrier()

        def chunk(i, c2):
            base = wid * epw + i * CH
            pltpu.sync_copy(row_hbm.at[pl.ds(base, CH)], idxv)
            pltpu.sync_copy(vals_hbm.at[pl.ds(base, CH)], buf)
            pltpu.sync_copy(buf, acc.at[idxv], add=True)
            return c2

        lax.fori_loop(0, n_chunks, chunk, 0)
        plsc.subcore_barrier()
        pltpu.sync_copy(acc.at[pl.ds(s * _NPS_P, _NPS_P)],
                        out_hbm.at[c, pl.ds(s * _NPS_P, _NPS_P)])

    return k(vals, row)


_BN = 1000
_BE = 2000


def _tc_proj_pair(h, Wa, Wb):
    def body(h_ref, wa_ref, wb_ref, a_ref, b_ref):
        hb = h_ref[...]
        a_ref[...] = jnp.dot(hb, wa_ref[...], preferred_element_type=_f32)
        b_ref[...] = jnp.dot(hb, wb_ref[...], preferred_element_type=_f32)

    grid = N // _BN
    return pl.pallas_call(
        body,
        grid=(grid,),
        in_specs=[
            pl.BlockSpec((_BN, H), lambda i: (i, 0)),
            pl.BlockSpec((H, H), lambda i: (0, 0)),
            pl.BlockSpec((H, H), lambda i: (0, 0)),
        ],
        out_specs=[
            pl.BlockSpec((_BN, H), lambda i: (i, 0)),
            pl.BlockSpec((_BN, H), lambda i: (i, 0)),
        ],
        out_shape=[
            jax.ShapeDtypeStruct((N, H), _f32),
            jax.ShapeDtypeStruct((N, H), _f32),
        ],
    )(h, Wa, Wb)


def _tc_edge_mlp(pre, diff, eattr, w_r, w_e, eb1, eW2, eb2, aWt, ab, off=0):
    ne = pre.shape[0]
    ob = off // _BE
    def body(pre_ref, d_ref, ea_ref, wr_ref, we_ref, b1_ref, w2_ref, b2_ref,
             awt_ref, ab_ref, ef_ref):
        d = d_ref[...]
        radial = jnp.sum(d * d, axis=1, keepdims=True)
        z1 = (pre_ref[...] + radial * wr_ref[...] + ea_ref[...] * we_ref[...]
              + b1_ref[...])
        u = _silu(z1)
        z2 = jnp.dot(u, w2_ref[...], preferred_element_type=_f32) + b2_ref[...]
        mij = _silu(z2)
        att = jax.nn.sigmoid(
            jnp.sum(mij * awt_ref[...], axis=1, keepdims=True) + ab_ref[...])
        ef_ref[...] = mij * att

    grid = ne // _BE
    return pl.pallas_call(
        body,
        grid=(grid,),
        in_specs=[
            pl.BlockSpec((_BE, H), lambda i: (i, 0)),
            pl.BlockSpec((_BE, XW), lambda i: (i + ob, 0)),
            pl.BlockSpec((_BE, 1), lambda i: (i + ob, 0)),
            pl.BlockSpec((1, H), lambda i: (0, 0)),
            pl.BlockSpec((1, H), lambda i: (0, 0)),
            pl.BlockSpec((1, H), lambda i: (0, 0)),
            pl.BlockSpec((H, H), lambda i: (0, 0)),
            pl.BlockSpec((1, H), lambda i: (0, 0)),
            pl.BlockSpec((1, H), lambda i: (0, 0)),
            pl.BlockSpec((1, 1), lambda i: (0, 0)),
        ],
        out_specs=pl.BlockSpec((_BE, H), lambda i: (i, 0)),
        out_shape=jax.ShapeDtypeStruct((ne, H), _f32),
    )(pre, diff, eattr, w_r, w_e, eb1, eW2, eb2, aWt, ab)


def _tc_coord_mlp(pre, diff, eattr, w_r, w_e, eb1, eW2, eb2, w3t, off=0):
    ne = pre.shape[0]
    ob = off // _BE
    def body(pre_ref, d_ref, ea_ref, wr_ref, we_ref, b1_ref, w2_ref, b2_ref,
             w3t_ref, tr_ref):
        d = d_ref[...]
        radial = jnp.sum(d * d, axis=1, keepdims=True)
        z1 = (pre_ref[...] + radial * wr_ref[...] + ea_ref[...] * we_ref[...]
              + b1_ref[...])
        u = _silu(z1)
        z2 = jnp.dot(u, w2_ref[...], preferred_element_type=_f32) + b2_ref[...]
        mij = _silu(z2)
        t = jnp.sum(mij * w3t_ref[...], axis=1, keepdims=True)
        cd = d / (jnp.sqrt(radial + 1e-8) + 1.0)
        tr_ref[...] = cd * t

    grid = ne // _BE
    return pl.pallas_call(
        body,
        grid=(grid,),
        in_specs=[
            pl.BlockSpec((_BE, H), lambda i: (i, 0)),
            pl.BlockSpec((_BE, XW), lambda i: (i + ob, 0)),
            pl.BlockSpec((_BE, 1), lambda i: (i + ob, 0)),
            pl.BlockSpec((1, H), lambda i: (0, 0)),
            pl.BlockSpec((1, H), lambda i: (0, 0)),
            pl.BlockSpec((1, H), lambda i: (0, 0)),
            pl.BlockSpec((H, H), lambda i: (0, 0)),
            pl.BlockSpec((1, H), lambda i: (0, 0)),
            pl.BlockSpec((1, H), lambda i: (0, 0)),
        ],
        out_specs=pl.BlockSpec((_BE, XW), lambda i: (i, 0)),
        out_shape=jax.ShapeDtypeStruct((ne, XW), _f32),
    )(pre, diff, eattr, w_r, w_e, eb1, eW2, eb2, w3t)


def _tc_node_mlp(h, aggp, nW1, nb1, nW2, nb2, Wa_next, Wb_next):
    def body(h_ref, p_ref, w1_ref, b1_ref, w2_ref, b2_ref, wa_ref, wb_ref,
             hn_ref, a_ref, b_ref):
        hb = h_ref[...]
        agg = (p_ref[0] + p_ref[1] + p_ref[2] + p_ref[3]) * 0.01
        w1 = w1_ref[...]
        z = (jnp.dot(hb, w1[:H], preferred_element_type=_f32)
             + jnp.dot(agg, w1[H:], preferred_element_type=_f32)
             + b1_ref[...])
        u = _silu(z)
        hn = hb + jnp.dot(u, w2_ref[...], preferred_element_type=_f32) + b2_ref[...]
        hn_ref[...] = hn
        a_ref[...] = jnp.dot(hn, wa_ref[...], preferred_element_type=_f32)
        b_ref[...] = jnp.dot(hn, wb_ref[...], preferred_element_type=_f32)

    grid = N // _BN
    return pl.pallas_call(
        body,
        grid=(grid,),
        in_specs=[
            pl.BlockSpec((_BN, H), lambda i: (i, 0)),
            pl.BlockSpec((4, _BN, H), lambda i: (0, i, 0)),
            pl.BlockSpec((2 * H, H), lambda i: (0, 0)),
            pl.BlockSpec((1, H), lambda i: (0, 0)),
            pl.BlockSpec((H, H), lambda i: (0, 0)),
            pl.BlockSpec((1, H), lambda i: (0, 0)),
            pl.BlockSpec((H, H), lambda i: (0, 0)),
            pl.BlockSpec((H, H), lambda i: (0, 0)),
        ],
        out_specs=[
            pl.BlockSpec((_BN, H), lambda i: (i, 0)),
            pl.BlockSpec((_BN, H), lambda i: (i, 0)),
            pl.BlockSpec((_BN, H), lambda i: (i, 0)),
        ],
        out_shape=[
            jax.ShapeDtypeStruct((N, H), _f32),
            jax.ShapeDtypeStruct((N, H), _f32),
            jax.ShapeDtypeStruct((N, H), _f32),
        ],
    )(h, aggp, nW1, nb1, nW2, nb2, Wa_next, Wb_next)


def _tc_coord_update(x16, xaggp):
    def body(x_ref, p_ref, o_ref):
        o_ref[...] = x_ref[...] + (p_ref[0] + p_ref[1] + p_ref[2] + p_ref[3]) * 0.01

    grid = N // _BN
    return pl.pallas_call(
        body,
        grid=(grid,),
        in_specs=[
            pl.BlockSpec((_BN, XW), lambda i: (i, 0)),
            pl.BlockSpec((4, _BN, XW), lambda i: (0, i, 0)),
        ],
        out_specs=pl.BlockSpec((_BN, XW), lambda i: (i, 0)),
        out_shape=jax.ShapeDtypeStruct((N, XW), _f32),
    )(x16, xaggp)


_NSPLIT = 2
_ES = E // _NSPLIT


def _edge_stage(A, B, row, col, diff, ea, wr, we, eb1, eW2, eb2, aWt, ab):
    parts = []
    for s in range(_NSPLIT):
        off = s * _ES
        rs = row[off:off + _ES]
        cs = col[off:off + _ES]
        pre = _sc_gather_sum(A, B, rs, cs)
        ef = _tc_edge_mlp(pre, diff, ea, wr, we, eb1, eW2, eb2, aWt, ab,
                          off=off)
        parts.append(_sc_segment_sum(ef, rs, H, 200))
    return jnp.concatenate(parts, axis=0)


def _coord_stage(A, B, row, col, diff, ea, wr, we, eb1, eW2, eb2, w3t):
    parts = []
    for s in range(_NSPLIT):
        off = s * _ES
        rs = row[off:off + _ES]
        cs = col[off:off + _ES]
        pre = _sc_gather_sum(A, B, rs, cs)
        trans = _tc_coord_mlp(pre, diff, ea, wr, we, eb1, eW2, eb2, w3t,
                              off=off)
        parts.append(_sc_segment_sum(trans, rs, XW, 200))
    return jnp.concatenate(parts, axis=0)


def kernel(h, x, edge_index, edge_attr,
           gcl0_eW1, gcl0_eb1, gcl0_eW2, gcl0_eb2, gcl0_nW1, gcl0_nb1,
           gcl0_nW2, gcl0_nb2, gcl0_aW, gcl0_ab,
           gcl1_eW1, gcl1_eb1, gcl1_eW2, gcl1_eb2, gcl1_nW1, gcl1_nb1,
           gcl1_nW2, gcl1_nb2, gcl1_aW, gcl1_ab,
           eq_W1, eq_b1, eq_W2, eq_b2, eq_W3):
    row = edge_index[0]
    col = edge_index[1]
    x16 = jnp.pad(x, ((0, 0), (0, XW - 3)))
    ea = edge_attr

    def esplit(W1):
        return (W1[:H], W1[H:2 * H], W1[2 * H:2 * H + 1].reshape(1, H),
                W1[2 * H + 1:].reshape(1, H))

    Wa0, Wb0, wr0, we0 = esplit(gcl0_eW1)
    A0, B0 = _tc_proj_pair(h, Wa0, Wb0)
    diff = _sc_coord_diff(x16, row, col)
    agg0 = _edge_stage(A0, B0, row, col, diff, ea, wr0, we0,
                       gcl0_eb1.reshape(1, H), gcl0_eW2,
                       gcl0_eb2.reshape(1, H), gcl0_aW.reshape(1, H),
                       gcl0_ab.reshape(1, 1))
    Wa1, Wb1, wr1, we1 = esplit(gcl1_eW1)
    h1, A1, B1 = _tc_node_mlp(h, agg0, gcl0_nW1, gcl0_nb1.reshape(1, H),
                              gcl0_nW2, gcl0_nb2.reshape(1, H), Wa1, Wb1)

    agg1 = _edge_stage(A1, B1, row, col, diff, ea, wr1, we1,
                       gcl1_eb1.reshape(1, H), gcl1_eW2,
                       gcl1_eb2.reshape(1, H), gcl1_aW.reshape(1, H),
                       gcl1_ab.reshape(1, 1))
    Wa2, Wb2, wr2, we2 = esplit(eq_W1)
    h2, A2, B2 = _tc_node_mlp(h1, agg1, gcl1_nW1, gcl1_nb1.reshape(1, H),
                              gcl1_nW2, gcl1_nb2.reshape(1, H), Wa2, Wb2)

    xagg = _coord_stage(A2, B2, row, col, diff, ea, wr2, we2,
                        eq_b1.reshape(1, H), eq_W2, eq_b2.reshape(1, H),
                        eq_W3.reshape(1, H))
    x16_out = _tc_coord_update(x16, xagg)

    return (h2, x16_out[:, :3])


# speedup vs baseline: 1.2429x; 1.0675x over previous
import functools

import jax
import jax.numpy as jnp
from jax import lax
from jax.experimental import pallas as pl
from jax.experimental.pallas import tpu as pltpu
from jax.experimental.pallas import tpu_sc as plsc

N = 10000
E = 320000
H = 128
XW = 16

_NC = 2
_NS = 16
_NW = _NC * _NS
_EPW = E // _NW
_NPS = N // _NS

_f32 = jnp.float32


def _silu(v):
    return v * jax.nn.sigmoid(v)


def _sc_gather_sum(A, B, row, col):
    ne = row.shape[0]
    epw = ne // _NW
    CH = 200
    n_chunks = epw // CH

    mesh = plsc.VectorSubcoreMesh(core_axis_name="c", subcore_axis_name="s")

    @functools.partial(
        pl.kernel, mesh=mesh,
        out_type=jax.ShapeDtypeStruct((ne, H), _f32),
        scratch_types=[
            pltpu.VMEM((CH,), jnp.int32),
            pltpu.VMEM((CH,), jnp.int32),
            pltpu.VMEM((CH,), jnp.int32),
            pltpu.VMEM((CH,), jnp.int32),
            pltpu.VMEM((CH, H), _f32),
            pltpu.VMEM((CH, H), _f32),
            pltpu.VMEM((CH, H), _f32),
            pltpu.VMEM((CH, H), _f32),
            pltpu.SemaphoreType.DMA,
            pltpu.SemaphoreType.DMA,
            pltpu.SemaphoreType.DMA,
            pltpu.SemaphoreType.DMA,
        ],
    )
    def k(A_hbm, B_hbm, row_hbm, col_hbm, pre_hbm,
          idxr0, idxc0, idxr1, idxc1,
          bufA0, bufB0, bufA1, bufB1, semA0, semB0, semA1, semB1):
        wid = lax.axis_index("s") * _NC + lax.axis_index("c")
        base0 = wid * epw
        idxsR = (idxr0, idxr1)
        idxsC = (idxc0, idxc1)
        bufsA = (bufA0, bufA1)
        bufsB = (bufB0, bufB1)
        semsA = (semA0, semA1)
        semsB = (semB0, semB1)

        def issue(i, b):
            base = base0 + i * CH
            pltpu.sync_copy(row_hbm.at[pl.ds(base, CH)], idxsR[b])
            pltpu.sync_copy(col_hbm.at[pl.ds(base, CH)], idxsC[b])
            pltpu.async_copy(A_hbm.at[idxsR[b]], bufsA[b], semsA[b])
            pltpu.async_copy(B_hbm.at[idxsC[b]], bufsB[b], semsB[b])

        issue(0, 0)

        def outer(g, carry):
            for b in range(2):
                i = g + b
                nb = 1 - b

                @pl.when(i < n_chunks)
                def _():
                    @pl.when(i + 1 < n_chunks)
                    def _():
                        issue(i + 1, nb)

                    pltpu.make_async_copy(
                        A_hbm.at[idxsR[b]], bufsA[b], semsA[b]).wait()
                    pltpu.make_async_copy(
                        B_hbm.at[idxsC[b]], bufsB[b], semsB[b]).wait()

                    bA, bB = bufsA[b], bufsB[b]

                    def add_row(r, c2):
                        for kk in range(H // 16):
                            sl = pl.ds(kk * 16, 16)
                            bA[r, sl] = bA[r, sl] + bB[r, sl]
                        return c2

                    lax.fori_loop(0, CH, add_row, 0)
                    pltpu.sync_copy(bA,
                                    pre_hbm.at[pl.ds(base0 + i * CH, CH)])
            return carry

        lax.fori_loop(0, (n_chunks + 1) // 2, lambda g, c: outer(g * 2, c), 0)

    return k(A, B, row, col)


def _sc_coord_diff(x16, row, col):
    mesh = plsc.VectorSubcoreMesh(core_axis_name="c", subcore_axis_name="s")
    CH = 200
    n_chunks = _EPW // CH

    @functools.partial(
        pl.kernel, mesh=mesh,
        out_type=jax.ShapeDtypeStruct((E, XW), _f32),
        compiler_params=pltpu.CompilerParams(use_tc_tiling_on_sc=False),
        scratch_types=[
            pltpu.VMEM((CH,), jnp.int32),
            pltpu.VMEM((CH,), jnp.int32),
            pltpu.VMEM((CH, XW), _f32),
            pltpu.VMEM((CH, XW), _f32),
            pltpu.SemaphoreType.DMA,
            pltpu.SemaphoreType.DMA,
        ],
    )
    def k(x_hbm, row_hbm, col_hbm, diff_hbm,
          idxr, idxc, bufXa, bufXb, semXa, semXb):
        wid = lax.axis_index("s") * _NC + lax.axis_index("c")
        base0 = wid * _EPW

        def chunk(i, carry):
            base = base0 + i * CH
            pltpu.sync_copy(row_hbm.at[pl.ds(base, CH)], idxr)
            pltpu.sync_copy(col_hbm.at[pl.ds(base, CH)], idxc)
            cpXa = pltpu.async_copy(x_hbm.at[idxr], bufXa, semXa)
            cpXb = pltpu.async_copy(x_hbm.at[idxc], bufXb, semXb)
            cpXa.wait()
            cpXb.wait()

            def sub_row(r, c2):
                sl = pl.ds(0, 16)
                bufXa[r, sl] = bufXa[r, sl] - bufXb[r, sl]
                return c2

            lax.fori_loop(0, CH, sub_row, 0)
            pltpu.sync_copy(bufXa, diff_hbm.at[pl.ds(base, CH)])
            return carry

        lax.fori_loop(0, n_chunks, chunk, 0)

    return k(x16, row, col)


_NPAD = 10240
_NPS_P = _NPAD // _NS


def _sc_segment_sum(vals, row, F, CH):
    ne = vals.shape[0]
    epw = ne // _NW
    mesh = plsc.VectorSubcoreMesh(core_axis_name="c", subcore_axis_name="s")
    n_chunks = epw // CH
    ZR = 64
    params = (pltpu.CompilerParams(use_tc_tiling_on_sc=False)
              if F != H else None)

    @functools.partial(
        pl.kernel, mesh=mesh,
        compiler_params=params,
        out_type=jax.ShapeDtypeStruct((_NC, _NPAD, F), _f32),
        scratch_types=[
            pltpu.VMEM_SHARED((_NPAD, F), _f32),
            pltpu.VMEM((CH,), jnp.int32),
            pltpu.VMEM((CH, F), _f32),
            pltpu.VMEM((ZR, F), _f32),
        ],
    )
    def k(vals_hbm, row_hbm, out_hbm, acc, idxv, buf, bufZ):
        c = lax.axis_index("c")
        s = lax.axis_index("s")
        wid = s * _NC + c

        def zrow(r, c2):
            for kk in range(F // 16):
                bufZ[r, pl.ds(kk * 16, 16)] = jnp.zeros((16,), _f32)
            return c2

        lax.fori_loop(0, ZR, zrow, 0)

        def zcopy(m, c2):
            pltpu.sync_copy(bufZ, acc.at[pl.ds(s * _NPS_P + m * ZR, ZR)])
            return c2

        lax.fori_loop(0, _NPS_P // ZR, zcopy, 0)
        plsc.subcore_barrier()

        def chunk(i, c2):
            base = wid * epw + i * CH
            pltpu.sync_copy(row_hbm.at[pl.ds(base, CH)], idxv)
            pltpu.sync_copy(vals_hbm.at[pl.ds(base, CH)], buf)
            pltpu.sync_copy(buf, acc.at[idxv], add=True)
            return c2

        lax.fori_loop(0, n_chunks, chunk, 0)
        plsc.subcore_barrier()
        pltpu.sync_copy(acc.at[pl.ds(s * _NPS_P, _NPS_P)],
                        out_hbm.at[c, pl.ds(s * _NPS_P, _NPS_P)])

    return k(vals, row)


_BN = 1000
_BE = 2000


def _tc_proj_pair(h, Wa, Wb):
    def body(h_ref, wa_ref, wb_ref, a_ref, b_ref):
        hb = h_ref[...]
        a_ref[...] = jnp.dot(hb, wa_ref[...], preferred_element_type=_f32)
        b_ref[...] = jnp.dot(hb, wb_ref[...], preferred_element_type=_f32)

    grid = N // _BN
    return pl.pallas_call(
        body,
        grid=(grid,),
        in_specs=[
            pl.BlockSpec((_BN, H), lambda i: (i, 0)),
            pl.BlockSpec((H, H), lambda i: (0, 0)),
            pl.BlockSpec((H, H), lambda i: (0, 0)),
        ],
        out_specs=[
            pl.BlockSpec((_BN, H), lambda i: (i, 0)),
            pl.BlockSpec((_BN, H), lambda i: (i, 0)),
        ],
        out_shape=[
            jax.ShapeDtypeStruct((N, H), _f32),
            jax.ShapeDtypeStruct((N, H), _f32),
        ],
    )(h, Wa, Wb)


def _tc_edge_mlp(pre, diff, eattr, w_r, w_e, eb1, eW2, eb2, aWt, ab, off=0):
    ne = pre.shape[0]
    ob = off // _BE
    def body(pre_ref, d_ref, ea_ref, wr_ref, we_ref, b1_ref, w2_ref, b2_ref,
             awt_ref, ab_ref, ef_ref):
        d = d_ref[...]
        radial = jnp.sum(d * d, axis=1, keepdims=True)
        z1 = (pre_ref[...] + radial * wr_ref[...] + ea_ref[...] * we_ref[...]
              + b1_ref[...])
        u = _silu(z1)
        z2 = jnp.dot(u, w2_ref[...], preferred_element_type=_f32) + b2_ref[...]
        mij = _silu(z2)
        att = jax.nn.sigmoid(
            jnp.sum(mij * awt_ref[...], axis=1, keepdims=True) + ab_ref[...])
        ef_ref[...] = mij * att

    grid = ne // _BE
    return pl.pallas_call(
        body,
        grid=(grid,),
        in_specs=[
            pl.BlockSpec((_BE, H), lambda i: (i, 0)),
            pl.BlockSpec((_BE, XW), lambda i: (i + ob, 0)),
            pl.BlockSpec((_BE, 1), lambda i: (i + ob, 0)),
            pl.BlockSpec((1, H), lambda i: (0, 0)),
            pl.BlockSpec((1, H), lambda i: (0, 0)),
            pl.BlockSpec((1, H), lambda i: (0, 0)),
            pl.BlockSpec((H, H), lambda i: (0, 0)),
            pl.BlockSpec((1, H), lambda i: (0, 0)),
            pl.BlockSpec((1, H), lambda i: (0, 0)),
            pl.BlockSpec((1, 1), lambda i: (0, 0)),
        ],
        out_specs=pl.BlockSpec((_BE, H), lambda i: (i, 0)),
        out_shape=jax.ShapeDtypeStruct((ne, H), _f32),
    )(pre, diff, eattr, w_r, w_e, eb1, eW2, eb2, aWt, ab)


def _tc_coord_mlp(pre, diff, eattr, w_r, w_e, eb1, eW2, eb2, w3t, off=0):
    ne = pre.shape[0]
    ob = off // _BE
    def body(pre_ref, d_ref, ea_ref, wr_ref, we_ref, b1_ref, w2_ref, b2_ref,
             w3t_ref, tr_ref):
        d = d_ref[...]
        radial = jnp.sum(d * d, axis=1, keepdims=True)
        z1 = (pre_ref[...] + radial * wr_ref[...] + ea_ref[...] * we_ref[...]
              + b1_ref[...])
        u = _silu(z1)
        z2 = jnp.dot(u, w2_ref[...], preferred_element_type=_f32) + b2_ref[...]
        mij = _silu(z2)
        t = jnp.sum(mij * w3t_ref[...], axis=1, keepdims=True)
        cd = d / (jnp.sqrt(radial + 1e-8) + 1.0)
        tr_ref[...] = cd * t

    grid = ne // _BE
    return pl.pallas_call(
        body,
        grid=(grid,),
        in_specs=[
            pl.BlockSpec((_BE, H), lambda i: (i, 0)),
            pl.BlockSpec((_BE, XW), lambda i: (i + ob, 0)),
            pl.BlockSpec((_BE, 1), lambda i: (i + ob, 0)),
            pl.BlockSpec((1, H), lambda i: (0, 0)),
            pl.BlockSpec((1, H), lambda i: (0, 0)),
            pl.BlockSpec((1, H), lambda i: (0, 0)),
            pl.BlockSpec((H, H), lambda i: (0, 0)),
            pl.BlockSpec((1, H), lambda i: (0, 0)),
            pl.BlockSpec((1, H), lambda i: (0, 0)),
        ],
        out_specs=pl.BlockSpec((_BE, XW), lambda i: (i, 0)),
        out_shape=jax.ShapeDtypeStruct((ne, XW), _f32),
    )(pre, diff, eattr, w_r, w_e, eb1, eW2, eb2, w3t)


def _tc_node_mlp(h, aggp, nW1, nb1, nW2, nb2, Wa_next, Wb_next):
    def body(h_ref, p_ref, w1_ref, b1_ref, w2_ref, b2_ref, wa_ref, wb_ref,
             hn_ref, a_ref, b_ref):
        hb = h_ref[...]
        agg = (p_ref[0] + p_ref[1] + p_ref[2] + p_ref[3]) * 0.01
        w1 = w1_ref[...]
        z = (jnp.dot(hb, w1[:H], preferred_element_type=_f32)
             + jnp.dot(agg, w1[H:], preferred_element_type=_f32)
             + b1_ref[...])
        u = _silu(z)
        hn = hb + jnp.dot(u, w2_ref[...], preferred_element_type=_f32) + b2_ref[...]
        hn_ref[...] = hn
        a_ref[...] = jnp.dot(hn, wa_ref[...], preferred_element_type=_f32)
        b_ref[...] = jnp.dot(hn, wb_ref[...], preferred_element_type=_f32)

    grid = N // _BN
    return pl.pallas_call(
        body,
        grid=(grid,),
        in_specs=[
            pl.BlockSpec((_BN, H), lambda i: (i, 0)),
            pl.BlockSpec((4, _BN, H), lambda i: (0, i, 0)),
            pl.BlockSpec((2 * H, H), lambda i: (0, 0)),
            pl.BlockSpec((1, H), lambda i: (0, 0)),
            pl.BlockSpec((H, H), lambda i: (0, 0)),
            pl.BlockSpec((1, H), lambda i: (0, 0)),
            pl.BlockSpec((H, H), lambda i: (0, 0)),
            pl.BlockSpec((H, H), lambda i: (0, 0)),
        ],
        out_specs=[
            pl.BlockSpec((_BN, H), lambda i: (i, 0)),
            pl.BlockSpec((_BN, H), lambda i: (i, 0)),
            pl.BlockSpec((_BN, H), lambda i: (i, 0)),
        ],
        out_shape=[
            jax.ShapeDtypeStruct((N, H), _f32),
            jax.ShapeDtypeStruct((N, H), _f32),
            jax.ShapeDtypeStruct((N, H), _f32),
        ],
    )(h, aggp, nW1, nb1, nW2, nb2, Wa_next, Wb_next)


def _tc_coord_update(x16, xaggp):
    def body(x_ref, p_ref, o_ref):
        o_ref[...] = x_ref[...] + (p_ref[0] + p_ref[1] + p_ref[2] + p_ref[3]) * 0.01

    grid = N // _BN
    return pl.pallas_call(
        body,
        grid=(grid,),
        in_specs=[
            pl.BlockSpec((_BN, XW), lambda i: (i, 0)),
            pl.BlockSpec((4, _BN, XW), lambda i: (0, i, 0)),
        ],
        out_specs=pl.BlockSpec((_BN, XW), lambda i: (i, 0)),
        out_shape=jax.ShapeDtypeStruct((N, XW), _f32),
    )(x16, xaggp)


_NSPLIT = 2
_ES = E // _NSPLIT


def _edge_stage(A, B, row, col, diff, ea, wr, we, eb1, eW2, eb2, aWt, ab):
    parts = []
    for s in range(_NSPLIT):
        off = s * _ES
        rs = row[off:off + _ES]
        cs = col[off:off + _ES]
        pre = _sc_gather_sum(A, B, rs, cs)
        ef = _tc_edge_mlp(pre, diff, ea, wr, we, eb1, eW2, eb2, aWt, ab,
                          off=off)
        parts.append(_sc_segment_sum(ef, rs, H, 200))
    return jnp.concatenate(parts, axis=0)


def _coord_stage(A, B, row, col, diff, ea, wr, we, eb1, eW2, eb2, w3t):
    parts = []
    for s in range(_NSPLIT):
        off = s * _ES
        rs = row[off:off + _ES]
        cs = col[off:off + _ES]
        pre = _sc_gather_sum(A, B, rs, cs)
        trans = _tc_coord_mlp(pre, diff, ea, wr, we, eb1, eW2, eb2, w3t,
                              off=off)
        parts.append(_sc_segment_sum(trans, rs, XW, 200))
    return jnp.concatenate(parts, axis=0)


def kernel(h, x, edge_index, edge_attr,
           gcl0_eW1, gcl0_eb1, gcl0_eW2, gcl0_eb2, gcl0_nW1, gcl0_nb1,
           gcl0_nW2, gcl0_nb2, gcl0_aW, gcl0_ab,
           gcl1_eW1, gcl1_eb1, gcl1_eW2, gcl1_eb2, gcl1_nW1, gcl1_nb1,
           gcl1_nW2, gcl1_nb2, gcl1_aW, gcl1_ab,
           eq_W1, eq_b1, eq_W2, eq_b2, eq_W3):
    row = edge_index[0]
    col = edge_index[1]
    x16 = jnp.pad(x, ((0, 0), (0, XW - 3)))
    ea = edge_attr

    def esplit(W1):
        return (W1[:H], W1[H:2 * H], W1[2 * H:2 * H + 1].reshape(1, H),
                W1[2 * H + 1:].reshape(1, H))

    Wa0, Wb0, wr0, we0 = esplit(gcl0_eW1)
    A0, B0 = _tc_proj_pair(h, Wa0, Wb0)
    diff = _sc_coord_diff(x16, row, col)
    agg0 = _edge_stage(A0, B0, row, col, diff, ea, wr0, we0,
                       gcl0_eb1.reshape(1, H), gcl0_eW2,
                       gcl0_eb2.reshape(1, H), gcl0_aW.reshape(1, H),
                       gcl0_ab.reshape(1, 1))
    Wa1, Wb1, wr1, we1 = esplit(gcl1_eW1)
    h1, A1, B1 = _tc_node_mlp(h, agg0, gcl0_nW1, gcl0_nb1.reshape(1, H),
                              gcl0_nW2, gcl0_nb2.reshape(1, H), Wa1, Wb1)

    agg1 = _edge_stage(A1, B1, row, col, diff, ea, wr1, we1,
                       gcl1_eb1.reshape(1, H), gcl1_eW2,
                       gcl1_eb2.reshape(1, H), gcl1_aW.reshape(1, H),
                       gcl1_ab.reshape(1, 1))
    Wa2, Wb2, wr2, we2 = esplit(eq_W1)
    h2, A2, B2 = _tc_node_mlp(h1, agg1, gcl1_nW1, gcl1_nb1.reshape(1, H),
                              gcl1_nW2, gcl1_nb2.reshape(1, H), Wa2, Wb2)

    xagg = _coord_stage(A2, B2, row, col, diff, ea, wr2, we2,
                        eq_b1.reshape(1, H), eq_W2, eq_b2.reshape(1, H),
                        eq_W3.reshape(1, H))
    x16_out = _tc_coord_update(x16, xagg)

    return (h2, x16_out[:, :3])
